# baseline (speedup 1.0000x reference)
import functools

import jax
import jax.numpy as jnp
from jax.experimental import pallas as pl
from jax.experimental.pallas import tpu as pltpu

N_GRAPHS_C = 1024


def _gcn(x, src, dst, W, b, n, dinv):
    h = x @ W
    norm = dinv[src] * dinv[dst]
    msg = h[src] * norm[:, None]
    return jax.ops.segment_sum(msg, dst, num_segments=n) + b


def _gat(x, src, dst, W, a_src, a_dst, b, n):
    h = x @ W
    al_s = h @ a_src
    al_d = h @ a_dst
    e = jax.nn.leaky_relu(al_s[src] + al_d[dst], negative_slope=0.2)
    m = jax.ops.segment_max(e, dst, num_segments=n)
    ex = jnp.exp(e - m[dst])
    denom = jax.ops.segment_sum(ex, dst, num_segments=n)
    alpha = ex / (denom[dst] + 1e-16)
    return jax.ops.segment_sum(h[src] * alpha[:, None], dst, num_segments=n) + b


def _head_kernel(mol_feat_ref, prot_feat_ref, Wc1a_ref, Wc1b_ref, bc1_ref,
                 Wc2_ref, bc2_ref, out_ref):
    mol = mol_feat_ref[...]
    prot = prot_feat_ref[...]
    hid = mol @ Wc1a_ref[...] + prot @ Wc1b_ref[...] + bc1_ref[...]
    hid = jnp.maximum(hid, 0.0)
    out = hid @ Wc2_ref[...] + bc2_ref[...]
    out_ref[...] = jax.nn.sigmoid(out)


def _head(mol_feat, prot_feat, Wc1, bc1, Wc2, bc2):
    Wc1a = Wc1[:64]
    Wc1b = Wc1[64:]
    return pl.pallas_call(
        _head_kernel,
        out_shape=jax.ShapeDtypeStruct((N_GRAPHS_C, 1), jnp.float32),
    )(mol_feat, prot_feat[None, :], Wc1a, Wc1b, bc1[None, :], Wc2, bc2[None, :])


def kernel(mol_x, mol_edge_index, mol_batch, prot_x, prot_edge_index, W1, b1,
           W2, b2, Wg1, asrc1, adst1, bg1, Wg2, asrc2, adst2, bg2, Wc1, bc1,
           Wc2, bc2):
    n = mol_x.shape[0]
    loops = jnp.arange(n, dtype=mol_edge_index.dtype)
    src = jnp.concatenate([mol_edge_index[0], loops])
    dst = jnp.concatenate([mol_edge_index[1], loops])
    deg = jax.ops.segment_sum(jnp.ones_like(dst, dtype=jnp.float32), dst,
                              num_segments=n)
    dinv = jnp.where(deg > 0, 1.0 / jnp.sqrt(deg), 0.0)
    h = jax.nn.relu(_gcn(mol_x, src, dst, W1, b1, n, dinv))
    h = _gcn(h, src, dst, W2, b2, n, dinv)
    cnt = jax.ops.segment_sum(jnp.ones((n,), jnp.float32), mol_batch,
                              num_segments=N_GRAPHS_C)
    mol_feat = jax.ops.segment_sum(h, mol_batch, num_segments=N_GRAPHS_C)
    mol_feat = mol_feat / jnp.maximum(cnt, 1.0)[:, None]

    m = prot_x.shape[0]
    ploops = jnp.arange(m, dtype=prot_edge_index.dtype)
    psrc = jnp.concatenate([prot_edge_index[0], ploops])
    pdst = jnp.concatenate([prot_edge_index[1], ploops])
    hp = jax.nn.relu(_gat(prot_x, psrc, pdst, Wg1, asrc1, adst1, bg1, m))
    hp = _gat(hp, psrc, pdst, Wg2, asrc2, adst2, bg2, m)
    prot_feat = jnp.mean(hp, axis=0)

    return _head(mol_feat, prot_feat, Wc1, bc1, Wc2, bc2)


# speedup vs baseline: 1.2382x; 1.2382x over previous
import functools

import jax
import jax.numpy as jnp
from jax import lax
from jax.experimental import pallas as pl
from jax.experimental.pallas import tpu as pltpu
from jax.experimental.pallas import tpu_sc as plsc

N_MOL = 100000
E_MOL = 1600000
N_PROT = 50000
E_PROT = 1600000
N_GRAPHS = 1024

NC, NS = 2, 16
CH = 128
SB = 8
SUPER = SB * CH

F = 64
FQ = 16

N_MOLP = 100096
_WIN = N_MOLP // NS
_ZB = 368

_EDGES_PER_SC_PAD = -(-(E_MOL // NC) // (NS * SUPER)) * (NS * SUPER)
E_PAD = NC * _EDGES_PER_SC_PAD

_MESH = plsc.VectorSubcoreMesh(core_axis_name="c", subcore_axis_name="s",
                               num_cores=NC, num_subcores=NS)


_DEG_STEPS = _EDGES_PER_SC_PAD // (NS * SUPER)


@functools.partial(
    pl.kernel,
    out_type=jax.ShapeDtypeStruct((NC * N_MOLP,), jnp.float32),
    mesh=_MESH,
    compiler_params=pltpu.CompilerParams(use_tc_tiling_on_sc=False),
    scratch_types=[
        pltpu.VMEM_SHARED((N_MOLP,), jnp.float32),
        pltpu.VMEM((_WIN,), jnp.float32),
        pltpu.VMEM((CH,), jnp.float32),
        pltpu.VMEM((SB, CH), jnp.int32),
        pltpu.SemaphoreType.DMA,
        pltpu.SemaphoreType.DMA,
    ],
)
def _deg_kernel(dst_hbm, deg_out, spd, zbuf, ones_v, dstbuf, semi, sems):
    c = lax.axis_index("c")
    s = lax.axis_index("s")

    def zb(i, _):
        zbuf[pl.ds(i * 16, 16)] = jnp.zeros((16,), jnp.float32)
        return 0
    lax.fori_loop(0, _WIN // 16, zb, 0)
    pltpu.sync_copy(zbuf, spd.at[pl.ds(s * _WIN, _WIN)])

    def ob(i, _):
        ones_v[pl.ds(i * 16, 16)] = jnp.ones((16,), jnp.float32)
        return 0
    lax.fori_loop(0, CH // 16, ob, 0)
    plsc.subcore_barrier()

    def step(k, _):
        sc_id = k * NS + s
        base = c * _EDGES_PER_SC_PAD + sc_id * SUPER
        di = [pltpu.async_copy(dst_hbm.at[pl.ds(base + j * CH, CH)],
                               dstbuf.at[j], semi) for j in range(SB)]
        for d in di:
            d.wait()
        ds_ = [pltpu.async_copy(ones_v, spd.at[dstbuf.at[j]], sems, add=True)
               for j in range(SB)]
        for d in ds_:
            d.wait()
        return 0

    lax.fori_loop(0, _DEG_STEPS, step, 0)
    plsc.subcore_barrier()
    pltpu.sync_copy(spd.at[pl.ds(s * _WIN, _WIN)], zbuf)
    pltpu.sync_copy(zbuf, deg_out.at[pl.ds(c * N_MOLP + s * _WIN, _WIN)])


_GCN_STEPS = E_PAD // (NS * SUPER)


@functools.partial(
    pl.kernel,
    out_type=jax.ShapeDtypeStruct((4 * N_MOLP, FQ), jnp.float32),
    mesh=_MESH,
    compiler_params=pltpu.CompilerParams(use_tc_tiling_on_sc=False),
    scratch_types=[
        pltpu.VMEM_SHARED((N_MOLP, FQ), jnp.float32),
        pltpu.VMEM((_ZB, FQ), jnp.float32),
        pltpu.VMEM((SUPER,), jnp.int32),
        pltpu.VMEM((SB, CH), jnp.int32),
        pltpu.VMEM((SB, CH, FQ), jnp.float32),
        pltpu.SemaphoreType.DMA,
        pltpu.SemaphoreType.DMA,
        pltpu.SemaphoreType.DMA,
    ],
)
def _gcn_agg_kernel(src_hbm, dst_hbm, hs_flat, out_flat,
                    spm, zbuf, srcbuf, dstbuf, rows, semi, semg, sems):
    c = lax.axis_index("c")
    s = lax.axis_index("s")

    def zb(i, _):
        zbuf[i, :] = jnp.zeros((FQ,), jnp.float32)
        return 0

    for p in range(2):
        q = 2 * c + p
        lax.fori_loop(0, _ZB, zb, 0)
        for t in range(_WIN // _ZB):
            pltpu.sync_copy(zbuf, spm.at[pl.ds(s * _WIN + t * _ZB, _ZB)])
        plsc.subcore_barrier()

        off = q * N_MOLP

        def step(k, _):
            sc_id = k * NS + s
            base = sc_id * SUPER
            di = [pltpu.async_copy(src_hbm.at[pl.ds(base, SUPER)], srcbuf,
                                   semi)]
            di += [pltpu.async_copy(dst_hbm.at[pl.ds(base + j * CH, CH)],
                                    dstbuf.at[j], semi) for j in range(SB)]
            for d in di:
                d.wait()

            def adj(g, _):
                srcbuf[pl.ds(g * 16, 16)] = srcbuf[pl.ds(g * 16, 16)] + off
                return 0
            lax.fori_loop(0, SUPER // 16, adj, 0)

            dg = [pltpu.async_copy(hs_flat.at[srcbuf.at[pl.ds(j * CH, CH)]],
                                   rows.at[j], semg) for j in range(SB)]
            for d in dg:
                d.wait()
            ds_ = [pltpu.async_copy(rows.at[j], spm.at[dstbuf.at[j]], sems,
                                    add=True) for j in range(SB)]
            for d in ds_:
                d.wait()
            return 0

        lax.fori_loop(0, _GCN_STEPS, step, 0)
        plsc.subcore_barrier()
        for t in range(_WIN // _ZB):
            r0 = s * _WIN + t * _ZB
            pltpu.sync_copy(spm.at[pl.ds(r0, _ZB)], zbuf)
            pltpu.sync_copy(zbuf, out_flat.at[pl.ds(q * N_MOLP + r0, _ZB)])
        plsc.subcore_barrier()


def _gcn_layer_agg(src_pad, dst_pad, hs):
    hs4 = hs.reshape(N_MOL, 4, FQ).transpose(1, 0, 2)
    hs4 = jnp.concatenate(
        [hs4, jnp.zeros((4, N_MOLP - N_MOL, FQ), jnp.float32)], axis=1)
    out = _gcn_agg_kernel(src_pad, dst_pad, hs4.reshape(4 * N_MOLP, FQ))
    out = out.reshape(4, N_MOLP, FQ)[:, :N_MOL, :]
    return out.transpose(1, 0, 2).reshape(N_MOL, F)


def _gat(x, src, dst, W, a_src, a_dst, b, n):
    h = x @ W
    al_s = h @ a_src
    al_d = h @ a_dst
    e = jax.nn.leaky_relu(al_s[src] + al_d[dst], negative_slope=0.2)
    m = jax.ops.segment_max(e, dst, num_segments=n)
    ex = jnp.exp(e - m[dst])
    denom = jax.ops.segment_sum(ex, dst, num_segments=n)
    alpha = ex / (denom[dst] + 1e-16)
    return jax.ops.segment_sum(h[src] * alpha[:, None], dst, num_segments=n) + b


def _head_kernel(mol_feat_ref, prot_feat_ref, Wc1a_ref, Wc1b_ref, bc1_ref,
                 Wc2_ref, bc2_ref, out_ref):
    mol = mol_feat_ref[...]
    prot = prot_feat_ref[...]
    hid = mol @ Wc1a_ref[...] + prot @ Wc1b_ref[...] + bc1_ref[...]
    hid = jnp.maximum(hid, 0.0)
    out = hid @ Wc2_ref[...] + bc2_ref[...]
    out_ref[...] = jax.nn.sigmoid(out)


def _head(mol_feat, prot_feat, Wc1, bc1, Wc2, bc2):
    return pl.pallas_call(
        _head_kernel,
        out_shape=jax.ShapeDtypeStruct((N_GRAPHS, 1), jnp.float32),
    )(mol_feat, prot_feat[None, :], Wc1[:64], Wc1[64:], bc1[None, :],
      Wc2, bc2[None, :])


def kernel(mol_x, mol_edge_index, mol_batch, prot_x, prot_edge_index, W1, b1,
           W2, b2, Wg1, asrc1, adst1, bg1, Wg2, asrc2, adst2, bg2, Wc1, bc1,
           Wc2, bc2):
    src = mol_edge_index[0]
    dst = mol_edge_index[1]
    pad = E_PAD - E_MOL
    src_pad = jnp.concatenate([src, jnp.zeros((pad,), jnp.int32)])
    dst_pad = jnp.concatenate([dst, jnp.full((pad,), N_MOL, jnp.int32)])

    deg_parts = _deg_kernel(dst_pad).reshape(NC, N_MOLP)
    deg = deg_parts[0, :N_MOL] + deg_parts[1, :N_MOL] + 1.0
    dinv = lax.rsqrt(deg)

    h = mol_x @ W1
    hs = h * dinv[:, None]
    A = _gcn_layer_agg(src_pad, dst_pad, hs)
    h1 = jax.nn.relu(dinv[:, None] * (A + hs) + b1)
    h2m = h1 @ W2
    hs2 = h2m * dinv[:, None]
    A2 = _gcn_layer_agg(src_pad, dst_pad, hs2)
    h2 = dinv[:, None] * (A2 + hs2) + b2

    cnt = jax.ops.segment_sum(jnp.ones((N_MOL,), jnp.float32), mol_batch,
                              num_segments=N_GRAPHS)
    mol_feat = jax.ops.segment_sum(h2, mol_batch, num_segments=N_GRAPHS)
    mol_feat = mol_feat / jnp.maximum(cnt, 1.0)[:, None]

    m = prot_x.shape[0]
    ploops = jnp.arange(m, dtype=prot_edge_index.dtype)
    psrc = jnp.concatenate([prot_edge_index[0], ploops])
    pdst = jnp.concatenate([prot_edge_index[1], ploops])
    hp = jax.nn.relu(_gat(prot_x, psrc, pdst, Wg1, asrc1, adst1, bg1, m))
    hp = _gat(hp, psrc, pdst, Wg2, asrc2, adst2, bg2, m)
    prot_feat = jnp.mean(hp, axis=0)

    return _head(mol_feat, prot_feat, Wc1, bc1, Wc2, bc2)
